# Initial kernel scaffold; baseline (speedup 1.0000x reference)
import jax
import jax.numpy as jnp
from jax.experimental import pallas as pl


def kernel(x, edge_index, adj0_vals, adj1_vals, kernel):
    raise NotImplementedError("write your pallas kernel here")


# speedup vs baseline: 3.0827x; 3.0827x over previous
import functools

import jax
import jax.numpy as jnp
from jax import lax
from jax.experimental import pallas as pl
from jax.experimental.pallas import tpu as pltpu
from jax.experimental.pallas import tpu_sc as plsc

N = 10000
D = 128
E = 320000
NS = 16
CHUNK = 80
EPT = E // NS
NCHUNK = EPT // CHUNK
ROWS_PT = 624
TAIL = N - NS * ROWS_PT
RB = 1000


def _dense_body(x_ref, w_ref, p_ref, q_ref):
    h = jnp.dot(x_ref[...], w_ref[...], preferred_element_type=jnp.float32)
    r = jnp.maximum(h, 0.0)
    att = jnp.exp(-r)
    m = jnp.where(h > 0, h, jnp.exp(h) - 1.0)
    p_ref[...] = m * att
    q_ref[...] = r * att * att


def _dense(x, w):
    return pl.pallas_call(
        _dense_body,
        grid=(N // RB,),
        in_specs=[
            pl.BlockSpec((RB, D), lambda i: (i, 0)),
            pl.BlockSpec((D, D), lambda i: (0, 0)),
        ],
        out_specs=[
            pl.BlockSpec((RB, D), lambda i: (i, 0)),
            pl.BlockSpec((RB, D), lambda i: (i, 0)),
        ],
        out_shape=[
            jax.ShapeDtypeStruct((N, D), jnp.float32),
            jax.ShapeDtypeStruct((N, D), jnp.float32),
        ],
    )(x, w)


def _sc_body(row_hbm, col_hbm, vals_hbm, pq_hbm, zeros_hbm, out_hbm,
             idx_v, row_v, rows_v, val_v, acc, sem):
    c = lax.axis_index("c")
    s = lax.axis_index("s")
    pltpu.sync_copy(zeros_hbm.at[pl.ds(0, ROWS_PT)],
                    acc.at[pl.ds(s * ROWS_PT, ROWS_PT)])

    @pl.when(s == NS - 1)
    def _():
        pltpu.sync_copy(zeros_hbm.at[pl.ds(0, TAIL)],
                        acc.at[pl.ds(NS * ROWS_PT, TAIL)])

    plsc.subcore_barrier()
    cN = c * N
    base = s * EPT

    def chunk_body(j, carry):
        off = pl.multiple_of(base + j * CHUNK, CHUNK)
        pltpu.sync_copy(col_hbm.at[pl.ds(off, CHUNK)], idx_v)
        pltpu.sync_copy(row_hbm.at[pl.ds(off, CHUNK)], row_v)
        pltpu.sync_copy(vals_hbm.at[pl.ds(c * E + off, CHUNK)], val_v)
        for i in range(CHUNK // 16):
            sl = pl.ds(i * 16, 16)
            idx_v[sl] = idx_v[sl] + cN
        pltpu.async_copy(pq_hbm.at[idx_v], rows_v, sem).wait()

        def grp_body(g, ecarry):
            vv = val_v[pl.ds(g * 16, 16)]
            for l in range(16):
                e = g * 16 + l
                v = vv[l]
                for f in range(D // 16):
                    sl = pl.ds(f * 16, 16)
                    rows_v[e, sl] = rows_v[e, sl] * v
            return ecarry

        lax.fori_loop(0, CHUNK // 16, grp_body, 0)
        pltpu.sync_copy(rows_v, acc.at[row_v], add=True)
        return carry

    lax.fori_loop(0, NCHUNK, chunk_body, 0)
    plsc.subcore_barrier()
    r0 = s * ROWS_PT
    pltpu.sync_copy(acc.at[pl.ds(r0, ROWS_PT)],
                    out_hbm.at[pl.ds(cN + r0, ROWS_PT)])

    @pl.when(s == NS - 1)
    def _():
        pltpu.sync_copy(acc.at[pl.ds(NS * ROWS_PT, TAIL)],
                        out_hbm.at[pl.ds(cN + NS * ROWS_PT, TAIL)])


_sc_call = functools.partial(
    pl.kernel,
    mesh=plsc.VectorSubcoreMesh(core_axis_name="c", subcore_axis_name="s"),
    out_type=jax.ShapeDtypeStruct((2 * N, D), jnp.float32),
    scratch_types=[
        pltpu.VMEM((CHUNK,), jnp.int32),
        pltpu.VMEM((CHUNK,), jnp.int32),
        pltpu.VMEM((CHUNK, D), jnp.float32),
        pltpu.VMEM((CHUNK,), jnp.float32),
        pltpu.VMEM_SHARED((N, D), jnp.float32),
        pltpu.SemaphoreType.DMA,
    ],
)(_sc_body)


def kernel(x, edge_index, adj0_vals, adj1_vals, kernel):
    p, q = _dense(x, kernel)
    pq = jnp.concatenate([p, q], axis=0)
    vals = jnp.concatenate([adj0_vals, adj1_vals])
    row = edge_index[0]
    col = edge_index[1]
    zeros = jnp.zeros((ROWS_PT, D), jnp.float32)
    out = _sc_call(row, col, vals, pq, zeros)
    return out[:N], out[N:]
